# baseline (speedup 1.0000x reference)
import functools

import jax
import jax.numpy as jnp
from jax import lax
from jax.experimental import pallas as pl
from jax.experimental.pallas import tpu as pltpu
from jax.experimental.pallas import tpu_sc as plsc

NC = 2
NS = 16
NW = NC * NS
L = 16


def _vgather(v, idx):
    dn = lax.GatherDimensionNumbers(
        offset_dims=(), collapsed_slice_dims=(0,), start_index_map=(0,))
    return lax.gather(v, idx.reshape(L, 1), dn, (1,),
                      mode=lax.GatherScatterMode.PROMISE_IN_BOUNDS)


def _sc_sample(verts_flat, table, B, N, H, W, D):
    BN = B * N
    vpw = BN // NW
    n_iters = vpw // L
    VCH = 32
    n_chunks = vpw // VCH

    mesh = plsc.VectorSubcoreMesh(
        core_axis_name="c", subcore_axis_name="s", num_cores=NC,
        num_subcores=NS)

    @functools.partial(
        pl.kernel,
        out_type=jax.ShapeDtypeStruct((BN * D,), jnp.float32),
        mesh=mesh,
        scratch_types=[
            pltpu.VMEM((vpw * 2,), jnp.float32),
            pltpu.VMEM((4, vpw), jnp.int32),
            pltpu.VMEM((4, vpw), jnp.float32),
            pltpu.VMEM((4, VCH, D), jnp.float32),
            pltpu.VMEM((VCH * D,), jnp.float32),
            pltpu.SemaphoreType.DMA,
        ],
        compiler_params=pltpu.CompilerParams(use_tc_tiling_on_sc=False),
    )
    def k(verts_hbm, table_hbm, out_hbm, verts_v, idx_v, wgt_v, rows_v,
          feats_v, sem):
        wid = lax.axis_index("s") * NC + lax.axis_index("c")
        vbase = wid * vpw
        base_row = (vbase // N) * (H * W)

        pltpu.sync_copy(verts_hbm.at[pl.ds(vbase * 2, vpw * 2)], verts_v)

        lane = lax.iota(jnp.int32, L)
        ia = (2 * lane) % L
        sel = lane < 8

        def idx_body(i, _):
            off = pl.multiple_of(i * (2 * L), 2 * L)
            va = verts_v[pl.ds(off, L)]
            vb = verts_v[pl.ds(off + L, L)]
            vy = jnp.where(sel, _vgather(va, ia), _vgather(vb, ia))
            vx = jnp.where(sel, _vgather(va, ia + 1), _vgather(vb, ia + 1))
            y = (vy + 1.0) * ((H - 1) * 0.5)
            x = (vx + 1.0) * ((W - 1) * 0.5)
            y = jnp.clip(y, 0.0, float(H - 1))
            x = jnp.clip(x, 0.0, float(W - 1))
            y0 = jnp.minimum(y.astype(jnp.int32), H - 2)
            x0 = jnp.minimum(x.astype(jnp.int32), W - 2)
            fy = y - y0.astype(jnp.float32)
            fx = x - x0.astype(jnp.float32)
            r00 = base_row + y0 * W + x0
            voff = pl.multiple_of(i * L, L)
            vsl = pl.ds(voff, L)
            idx_v[0, vsl] = r00
            idx_v[1, vsl] = r00 + 1
            idx_v[2, vsl] = r00 + W
            idx_v[3, vsl] = r00 + W + 1
            gy = 1.0 - fy
            gx = 1.0 - fx
            wgt_v[0, vsl] = gy * gx
            wgt_v[1, vsl] = gy * fx
            wgt_v[2, vsl] = fy * gx
            wgt_v[3, vsl] = fy * fx
            return 0

        lax.fori_loop(0, n_iters, idx_body, 0)

        def chunk_body(g, _):
            goff = pl.multiple_of(g * VCH, VCH)
            for kk in range(4):
                pltpu.async_copy(
                    table_hbm.at[idx_v.at[kk, pl.ds(goff, VCH)]],
                    rows_v.at[kk], sem)
            for kk in range(4):
                pltpu.make_async_copy(
                    table_hbm.at[idx_v.at[kk, pl.ds(goff, VCH)]],
                    rows_v.at[kk], sem).wait()

            def group_body(q, _):
                qoff = pl.multiple_of(q * L, L)
                w0 = wgt_v[0, pl.ds(goff + qoff, L)]
                w1 = wgt_v[1, pl.ds(goff + qoff, L)]
                w2 = wgt_v[2, pl.ds(goff + qoff, L)]
                w3 = wgt_v[3, pl.ds(goff + qoff, L)]
                for j in range(L):
                    jv = jnp.full((L,), j, jnp.int32)
                    b0 = _vgather(w0, jv)
                    b1 = _vgather(w1, jv)
                    b2 = _vgather(w2, jv)
                    b3 = _vgather(w3, jv)
                    v = qoff + j
                    for s in range(D // L):
                        sl = pl.ds(s * L, L)
                        acc = b0 * rows_v[0, v, sl]
                        acc += b1 * rows_v[1, v, sl]
                        acc += b2 * rows_v[2, v, sl]
                        acc += b3 * rows_v[3, v, sl]
                        feats_v[pl.ds(v * D + s * L, L)] = acc
                return 0

            lax.fori_loop(0, VCH // L, group_body, 0)
            obase = pl.multiple_of((vbase + g * VCH) * D, VCH * D)
            pltpu.sync_copy(feats_v, out_hbm.at[pl.ds(obase, VCH * D)])
            return 0

        lax.fori_loop(0, n_chunks, chunk_body, 0)

    return k(verts_flat, table)


def _mlp_body(x_ref, w1_ref, b1_ref, w2_ref, out_ref):
    h = jnp.dot(x_ref[...], w1_ref[...], preferred_element_type=jnp.float32)
    h = jnp.maximum(h + b1_ref[...], 0.0)
    out_ref[...] = jnp.dot(h, w2_ref[...], preferred_element_type=jnp.float32)


def _tc_mlp(feats, W1m, b1, W2m):
    BN, D = feats.shape
    DH = W1m.shape[1]
    DO = W2m.shape[1]
    BLK = 2048
    grid = (BN // BLK,)
    return pl.pallas_call(
        _mlp_body,
        grid=grid,
        in_specs=[
            pl.BlockSpec((BLK, D), lambda i: (i, 0)),
            pl.BlockSpec((D, DH), lambda i: (0, 0)),
            pl.BlockSpec((1, DH), lambda i: (0, 0)),
            pl.BlockSpec((DH, DO), lambda i: (0, 0)),
        ],
        out_specs=pl.BlockSpec((BLK, DO), lambda i: (i, 0)),
        out_shape=jax.ShapeDtypeStruct((BN, DO), jnp.float32),
    )(feats, W1m, b1.reshape(1, DH), W2m)


def kernel(vertices, feature_map, W1, b1, W2):
    B, N, _ = vertices.shape
    _, H, W, D = feature_map.shape
    table = feature_map.reshape(B * H * W, D)
    verts_flat = vertices.reshape(B * N * 2)
    feats = _sc_sample(verts_flat, table, B, N, H, W, D)
    feats = feats.reshape(B * N, D)
    out = _tc_mlp(feats, W1[0], b1, W2[0])
    return out.reshape(B, N, 2)


# speedup vs baseline: 1.0796x; 1.0796x over previous
import functools

import jax
import jax.numpy as jnp
from jax import lax
from jax.experimental import pallas as pl
from jax.experimental.pallas import tpu as pltpu
from jax.experimental.pallas import tpu_sc as plsc

NC = 2
NS = 16
NW = NC * NS
L = 16
TW = 256


def _vgather(v, idx):
    dn = lax.GatherDimensionNumbers(
        offset_dims=(), collapsed_slice_dims=(0,), start_index_map=(0,))
    return lax.gather(v, idx.reshape(L, 1), dn, (1,),
                      mode=lax.GatherScatterMode.PROMISE_IN_BOUNDS)


def _pad_body(x_ref, out_ref):
    out_ref[:, :x_ref.shape[1]] = x_ref[...]
    out_ref[:, x_ref.shape[1]:] = jnp.zeros(
        (x_ref.shape[0], out_ref.shape[1] - x_ref.shape[1]), jnp.float32)


def _tc_pad(table, D):
    V = table.shape[0]
    BLKR = 2048
    return pl.pallas_call(
        _pad_body,
        grid=(V // BLKR,),
        in_specs=[pl.BlockSpec((BLKR, D), lambda i: (i, 0))],
        out_specs=pl.BlockSpec((BLKR, TW), lambda i: (i, 0)),
        out_shape=jax.ShapeDtypeStruct((V, TW), jnp.float32),
    )(table)


def _sc_sample(verts_flat, table, B, N, H, W, D):
    BN = B * N
    vpw = BN // NW
    n_iters = vpw // L
    VCH = 32
    n_chunks = vpw // VCH

    mesh = plsc.VectorSubcoreMesh(
        core_axis_name="c", subcore_axis_name="s", num_cores=NC,
        num_subcores=NS)

    @functools.partial(
        pl.kernel,
        out_type=jax.ShapeDtypeStruct((BN, D), jnp.float32),
        mesh=mesh,
        scratch_types=[
            pltpu.VMEM((vpw * 2,), jnp.float32),
            pltpu.VMEM((4, vpw), jnp.int32),
            pltpu.VMEM((4, vpw), jnp.float32),
            pltpu.VMEM((4, VCH, TW), jnp.float32),
            pltpu.VMEM((VCH, D), jnp.float32),
            pltpu.SemaphoreType.DMA,
        ],
    )
    def k(verts_hbm, table_hbm, out_hbm, verts_v, idx_v, wgt_v, rows_v,
          feats_v, sem):
        wid = lax.axis_index("s") * NC + lax.axis_index("c")
        vbase = wid * vpw
        base_row = (vbase // N) * (H * W)

        pltpu.sync_copy(verts_hbm.at[pl.ds(vbase * 2, vpw * 2)], verts_v)

        lane = lax.iota(jnp.int32, L)
        ia = (2 * lane) % L
        sel = lane < 8

        def idx_body(i, _):
            off = pl.multiple_of(i * (2 * L), 2 * L)
            va = verts_v[pl.ds(off, L)]
            vb = verts_v[pl.ds(off + L, L)]
            vy = jnp.where(sel, _vgather(va, ia), _vgather(vb, ia))
            vx = jnp.where(sel, _vgather(va, ia + 1), _vgather(vb, ia + 1))
            y = (vy + 1.0) * ((H - 1) * 0.5)
            x = (vx + 1.0) * ((W - 1) * 0.5)
            y = jnp.clip(y, 0.0, float(H - 1))
            x = jnp.clip(x, 0.0, float(W - 1))
            y0 = jnp.minimum(y.astype(jnp.int32), H - 2)
            x0 = jnp.minimum(x.astype(jnp.int32), W - 2)
            fy = y - y0.astype(jnp.float32)
            fx = x - x0.astype(jnp.float32)
            r00 = base_row + y0 * W + x0
            voff = pl.multiple_of(i * L, L)
            vsl = pl.ds(voff, L)
            idx_v[0, vsl] = r00
            idx_v[1, vsl] = r00 + 1
            idx_v[2, vsl] = r00 + W
            idx_v[3, vsl] = r00 + W + 1
            gy = 1.0 - fy
            gx = 1.0 - fx
            wgt_v[0, vsl] = gy * gx
            wgt_v[1, vsl] = gy * fx
            wgt_v[2, vsl] = fy * gx
            wgt_v[3, vsl] = fy * fx
            return 0

        lax.fori_loop(0, n_iters, idx_body, 0)

        def chunk_body(g, _):
            goff = pl.multiple_of(g * VCH, VCH)
            for kk in range(4):
                pltpu.async_copy(
                    table_hbm.at[idx_v.at[kk, pl.ds(goff, VCH)]],
                    rows_v.at[kk], sem)
            for kk in range(4):
                pltpu.make_async_copy(
                    table_hbm.at[idx_v.at[kk, pl.ds(goff, VCH)]],
                    rows_v.at[kk], sem).wait()

            def group_body(q, _):
                qoff = pl.multiple_of(q * L, L)
                w0 = wgt_v[0, pl.ds(goff + qoff, L)]
                w1 = wgt_v[1, pl.ds(goff + qoff, L)]
                w2 = wgt_v[2, pl.ds(goff + qoff, L)]
                w3 = wgt_v[3, pl.ds(goff + qoff, L)]
                for j in range(L):
                    jv = jnp.full((L,), j, jnp.int32)
                    b0 = _vgather(w0, jv)
                    b1 = _vgather(w1, jv)
                    b2 = _vgather(w2, jv)
                    b3 = _vgather(w3, jv)
                    v = qoff + j
                    for s in range(D // L):
                        sl = pl.ds(s * L, L)
                        acc = b0 * rows_v[0, v, sl]
                        acc += b1 * rows_v[1, v, sl]
                        acc += b2 * rows_v[2, v, sl]
                        acc += b3 * rows_v[3, v, sl]
                        feats_v[v, sl] = acc
                return 0

            lax.fori_loop(0, VCH // L, group_body, 0)
            obase = pl.multiple_of(vbase + g * VCH, VCH)
            pltpu.sync_copy(feats_v, out_hbm.at[pl.ds(obase, VCH)])
            return 0

        lax.fori_loop(0, n_chunks, chunk_body, 0)

    return k(verts_flat, table)


def _mlp_body(x_ref, w1_ref, b1_ref, w2_ref, out_ref):
    h = jnp.dot(x_ref[...], w1_ref[...], preferred_element_type=jnp.float32)
    h = jnp.maximum(h + b1_ref[...], 0.0)
    out_ref[...] = jnp.dot(h, w2_ref[...], preferred_element_type=jnp.float32)


def _tc_mlp(feats, W1m, b1, W2m):
    BN, D = feats.shape
    DH = W1m.shape[1]
    DO = W2m.shape[1]
    BLK = 2048
    grid = (BN // BLK,)
    return pl.pallas_call(
        _mlp_body,
        grid=grid,
        in_specs=[
            pl.BlockSpec((BLK, D), lambda i: (i, 0)),
            pl.BlockSpec((D, DH), lambda i: (0, 0)),
            pl.BlockSpec((1, DH), lambda i: (0, 0)),
            pl.BlockSpec((DH, DO), lambda i: (0, 0)),
        ],
        out_specs=pl.BlockSpec((BLK, DO), lambda i: (i, 0)),
        out_shape=jax.ShapeDtypeStruct((BN, DO), jnp.float32),
    )(feats, W1m, b1.reshape(1, DH), W2m)


def kernel(vertices, feature_map, W1, b1, W2):
    B, N, _ = vertices.shape
    _, H, W, D = feature_map.shape
    table = _tc_pad(feature_map.reshape(B * H * W, D), D)
    verts_flat = vertices.reshape(B * N * 2)
    feats = _sc_sample(verts_flat, table, B, N, H, W, D)
    out = _tc_mlp(feats, W1[0], b1, W2[0])
    return out.reshape(B, N, 2)


# speedup vs baseline: 3.5674x; 3.3043x over previous
import functools

import jax
import jax.numpy as jnp
from jax import lax
from jax.experimental import pallas as pl
from jax.experimental.pallas import tpu as pltpu
from jax.experimental.pallas import tpu_sc as plsc

NC = 2
NS = 16
NW = NC * NS
L = 16
TW = 256


def _vgather(v, idx):
    dn = lax.GatherDimensionNumbers(
        offset_dims=(), collapsed_slice_dims=(0,), start_index_map=(0,))
    return lax.gather(v, idx.reshape(L, 1), dn, (1,),
                      mode=lax.GatherScatterMode.PROMISE_IN_BOUNDS)


def _repack_body(x_ref, out_ref):
    hb, D, W = x_ref.shape
    for h in range(hb):
        out_ref[pl.ds(h * W, W), :D] = x_ref[h].T
    out_ref[:, D:] = jnp.zeros((hb * W, TW - D), jnp.float32)


def _tc_repack(fm_t):
    BH, D, W = fm_t.shape
    HB = 8
    return pl.pallas_call(
        _repack_body,
        grid=(BH // HB,),
        in_specs=[pl.BlockSpec((HB, D, W), lambda i: (i, 0, 0))],
        out_specs=pl.BlockSpec((HB * W, TW), lambda i: (i, 0)),
        out_shape=jax.ShapeDtypeStruct((BH * W, TW), jnp.float32),
    )(fm_t)


def _sc_sample(verts_flat, table, B, N, H, W, D):
    BN = B * N
    vpw = BN // NW
    n_iters = vpw // L
    VCH = 32
    n_chunks = vpw // VCH

    mesh = plsc.VectorSubcoreMesh(
        core_axis_name="c", subcore_axis_name="s", num_cores=NC,
        num_subcores=NS)

    @functools.partial(
        pl.kernel,
        out_type=jax.ShapeDtypeStruct((BN, D), jnp.float32),
        mesh=mesh,
        scratch_types=[
            pltpu.VMEM((vpw * 2,), jnp.float32),
            pltpu.VMEM((4, vpw), jnp.int32),
            pltpu.VMEM((4, vpw), jnp.float32),
            pltpu.VMEM((4, VCH, TW), jnp.float32),
            pltpu.VMEM((VCH, D), jnp.float32),
            pltpu.SemaphoreType.DMA,
        ],
    )
    def k(verts_hbm, table_hbm, out_hbm, verts_v, idx_v, wgt_v, rows_v,
          feats_v, sem):
        wid = lax.axis_index("s") * NC + lax.axis_index("c")
        vbase = wid * vpw
        base_row = (vbase // N) * (H * W)

        pltpu.sync_copy(verts_hbm.at[pl.ds(vbase * 2, vpw * 2)], verts_v)

        lane = lax.iota(jnp.int32, L)
        ia = (2 * lane) % L
        sel = lane < 8

        def idx_body(i, _):
            off = pl.multiple_of(i * (2 * L), 2 * L)
            va = verts_v[pl.ds(off, L)]
            vb = verts_v[pl.ds(off + L, L)]
            vy = jnp.where(sel, _vgather(va, ia), _vgather(vb, ia))
            vx = jnp.where(sel, _vgather(va, ia + 1), _vgather(vb, ia + 1))
            y = (vy + 1.0) * ((H - 1) * 0.5)
            x = (vx + 1.0) * ((W - 1) * 0.5)
            y = jnp.clip(y, 0.0, float(H - 1))
            x = jnp.clip(x, 0.0, float(W - 1))
            y0 = jnp.minimum(y.astype(jnp.int32), H - 2)
            x0 = jnp.minimum(x.astype(jnp.int32), W - 2)
            fy = y - y0.astype(jnp.float32)
            fx = x - x0.astype(jnp.float32)
            r00 = base_row + y0 * W + x0
            voff = pl.multiple_of(i * L, L)
            vsl = pl.ds(voff, L)
            idx_v[0, vsl] = r00
            idx_v[1, vsl] = r00 + 1
            idx_v[2, vsl] = r00 + W
            idx_v[3, vsl] = r00 + W + 1
            gy = 1.0 - fy
            gx = 1.0 - fx
            wgt_v[0, vsl] = gy * gx
            wgt_v[1, vsl] = gy * fx
            wgt_v[2, vsl] = fy * gx
            wgt_v[3, vsl] = fy * fx
            return 0

        lax.fori_loop(0, n_iters, idx_body, 0)

        def chunk_body(g, _):
            goff = pl.multiple_of(g * VCH, VCH)
            for kk in range(4):
                pltpu.async_copy(
                    table_hbm.at[idx_v.at[kk, pl.ds(goff, VCH)]],
                    rows_v.at[kk], sem)
            for kk in range(4):
                pltpu.make_async_copy(
                    table_hbm.at[idx_v.at[kk, pl.ds(goff, VCH)]],
                    rows_v.at[kk], sem).wait()

            def group_body(q, _):
                qoff = pl.multiple_of(q * L, L)
                w0 = wgt_v[0, pl.ds(goff + qoff, L)]
                w1 = wgt_v[1, pl.ds(goff + qoff, L)]
                w2 = wgt_v[2, pl.ds(goff + qoff, L)]
                w3 = wgt_v[3, pl.ds(goff + qoff, L)]
                for j in range(L):
                    jv = jnp.full((L,), j, jnp.int32)
                    b0 = _vgather(w0, jv)
                    b1 = _vgather(w1, jv)
                    b2 = _vgather(w2, jv)
                    b3 = _vgather(w3, jv)
                    v = qoff + j
                    for s in range(D // L):
                        sl = pl.ds(s * L, L)
                        acc = b0 * rows_v[0, v, sl]
                        acc += b1 * rows_v[1, v, sl]
                        acc += b2 * rows_v[2, v, sl]
                        acc += b3 * rows_v[3, v, sl]
                        feats_v[v, sl] = acc
                return 0

            lax.fori_loop(0, VCH // L, group_body, 0)
            obase = pl.multiple_of(vbase + g * VCH, VCH)
            pltpu.sync_copy(feats_v, out_hbm.at[pl.ds(obase, VCH)])
            return 0

        lax.fori_loop(0, n_chunks, chunk_body, 0)

    return k(verts_flat, table)


def _mlp_body(x_ref, w1_ref, b1_ref, w2_ref, out_ref):
    h = jnp.dot(x_ref[...], w1_ref[...], preferred_element_type=jnp.float32)
    h = jnp.maximum(h + b1_ref[...], 0.0)
    out_ref[...] = jnp.dot(h, w2_ref[...], preferred_element_type=jnp.float32)


def _tc_mlp(feats, W1m, b1, W2m):
    BN, D = feats.shape
    DH = W1m.shape[1]
    DO = W2m.shape[1]
    BLK = 2048
    grid = (BN // BLK,)
    return pl.pallas_call(
        _mlp_body,
        grid=grid,
        in_specs=[
            pl.BlockSpec((BLK, D), lambda i: (i, 0)),
            pl.BlockSpec((D, DH), lambda i: (0, 0)),
            pl.BlockSpec((1, DH), lambda i: (0, 0)),
            pl.BlockSpec((DH, DO), lambda i: (0, 0)),
        ],
        out_specs=pl.BlockSpec((BLK, DO), lambda i: (i, 0)),
        out_shape=jax.ShapeDtypeStruct((BN, DO), jnp.float32),
    )(feats, W1m, b1.reshape(1, DH), W2m)


def kernel(vertices, feature_map, W1, b1, W2):
    B, N, _ = vertices.shape
    _, H, W, D = feature_map.shape
    fm_t = jnp.transpose(feature_map, (0, 1, 3, 2)).reshape(B * H, D, W)
    table = _tc_repack(fm_t)
    verts_flat = vertices.reshape(B * N * 2)
    feats = _sc_sample(verts_flat, table, B, N, H, W, D)
    out = _tc_mlp(feats, W1[0], b1, W2[0])
    return out.reshape(B, N, 2)
